# Initial kernel scaffold; baseline (speedup 1.0000x reference)
import jax
import jax.numpy as jnp
from jax.experimental import pallas as pl


def kernel(h, edge_index, W, b):
    raise NotImplementedError("write your pallas kernel here")


# speedup vs baseline: 7.8103x; 7.8103x over previous
import functools

import jax
import jax.numpy as jnp
from jax import lax
from jax.experimental import pallas as pl
from jax.experimental.pallas import tpu as pltpu
from jax.experimental.pallas import tpu_sc as plsc

N_NODES = 10000
N_EDGES = 320000
D = 128
NC, NS = 2, 16
NW = NC * NS
E_PER_TILE = N_EDGES // NW
CHUNK = 80
NCHUNK = E_PER_TILE // CHUNK
WB_TILES = 10
WB_ROWS = N_NODES // WB_TILES
ZR = 200


def _sc_segment_sum(src, dst, h):
    mesh = plsc.VectorSubcoreMesh(core_axis_name="c", subcore_axis_name="s")

    @functools.partial(
        pl.kernel,
        out_type=(
            jax.ShapeDtypeStruct((NC, N_NODES, D), jnp.float32),
            jax.ShapeDtypeStruct((NC * N_NODES,), jnp.float32),
        ),
        mesh=mesh,
        scratch_types=[
            pltpu.VMEM((CHUNK,), jnp.int32),
            pltpu.VMEM((CHUNK,), jnp.int32),
            pltpu.VMEM((CHUNK, D), jnp.float32),
            pltpu.VMEM((CHUNK,), jnp.float32),
            pltpu.VMEM((ZR, D), jnp.float32),
            pltpu.VMEM((WB_ROWS,), jnp.float32),
            pltpu.VMEM_SHARED((N_NODES, D), jnp.float32),
            pltpu.VMEM_SHARED((N_NODES,), jnp.float32),
            pltpu.SemaphoreType.DMA,
        ],
    )
    def body(src_hbm, dst_hbm, h_hbm, s_out, deg_out,
             sidx, didx, rows, ones, zbuf, dbuf, s_acc, d_acc, sem):
        c = lax.axis_index("c")
        s = lax.axis_index("s")
        zero16 = jnp.zeros((16,), jnp.float32)

        @pl.loop(0, ZR)
        def _zero_zbuf(r):
            for j in range(D // 16):
                zbuf[r, pl.ds(j * 16, 16)] = zero16

        @pl.loop(0, WB_ROWS // 16)
        def _zero_dbuf(r):
            dbuf[pl.ds(r * 16, 16)] = zero16

        for r in range(CHUNK // 16):
            ones[pl.ds(r * 16, 16)] = jnp.ones((16,), jnp.float32)

        tb = s * WB_ROWS

        @pl.when(s < WB_TILES)
        def _init():
            @pl.loop(0, WB_ROWS // ZR)
            def _zero_sacc(k):
                pltpu.sync_copy(zbuf, s_acc.at[pl.ds(tb + k * ZR, ZR)])
            pltpu.sync_copy(dbuf, d_acc.at[pl.ds(tb, WB_ROWS)])

        plsc.subcore_barrier()

        ebase = (c * NS + s) * E_PER_TILE

        @pl.loop(0, NCHUNK)
        def _chunk(i):
            base = ebase + i * CHUNK
            pltpu.sync_copy(src_hbm.at[pl.ds(base, CHUNK)], sidx)
            pltpu.sync_copy(dst_hbm.at[pl.ds(base, CHUNK)], didx)
            pltpu.async_copy(h_hbm.at[sidx], rows, sem).wait()
            pltpu.sync_copy(rows, s_acc.at[didx], add=True)
            pltpu.sync_copy(ones, d_acc.at[didx], add=True)

        plsc.subcore_barrier()

        @pl.when(s < WB_TILES)
        def _writeback():
            @pl.loop(0, WB_ROWS // ZR)
            def _wb(k):
                r0 = tb + k * ZR
                pltpu.sync_copy(s_acc.at[pl.ds(r0, ZR)], zbuf)
                pltpu.sync_copy(zbuf, s_out.at[c, pl.ds(r0, ZR)])
            pltpu.sync_copy(d_acc.at[pl.ds(tb, WB_ROWS)], dbuf)
            pltpu.sync_copy(dbuf, deg_out.at[pl.ds(c * N_NODES + tb, WB_ROWS)])

    return body(src, dst, h)


_BLK = 1000


def _tc_body(h_ref, s_ref, dg_ref, w1_ref, w2_ref, b_ref, o_ref):
    hv = h_ref[...]
    sv = s_ref[0] + s_ref[1]
    deg = dg_ref[0] + dg_ref[1]
    h_n = (sv + deg * hv) / jnp.maximum(deg, 1.0)
    o_ref[...] = (
        jnp.dot(hv, w1_ref[...], preferred_element_type=jnp.float32)
        + jnp.dot(h_n, w2_ref[...], preferred_element_type=jnp.float32)
        + b_ref[...]
    )


def _tc_combine(h, s_parts, deg_parts, w1t, w2t, b2d):
    grid = (N_NODES // _BLK,)
    return pl.pallas_call(
        _tc_body,
        grid=grid,
        in_specs=[
            pl.BlockSpec((_BLK, D), lambda i: (i, 0)),
            pl.BlockSpec((NC, _BLK, D), lambda i: (0, i, 0)),
            pl.BlockSpec((NC, _BLK, 1), lambda i: (0, i, 0)),
            pl.BlockSpec((D, D), lambda i: (0, 0)),
            pl.BlockSpec((D, D), lambda i: (0, 0)),
            pl.BlockSpec((1, D), lambda i: (0, 0)),
        ],
        out_specs=pl.BlockSpec((_BLK, D), lambda i: (i, 0)),
        out_shape=jax.ShapeDtypeStruct((N_NODES, D), jnp.float32),
    )(h, s_parts, deg_parts, w1t, w2t, b2d)


def kernel(h, edge_index, W, b):
    src = edge_index[0].astype(jnp.int32)
    dst = edge_index[1].astype(jnp.int32)
    s_parts, deg_parts = _sc_segment_sum(src, dst, h)
    wt = W.T
    return _tc_combine(h, s_parts, deg_parts.reshape(NC, N_NODES, 1),
                       wt[:D], wt[D:], b.reshape(1, D))
